# baseline (speedup 1.0000x reference)
import functools

import jax
import jax.numpy as jnp
from jax import lax
from jax.experimental import pallas as pl
from jax.experimental.pallas import tpu as pltpu
from jax.experimental.pallas import tpu_sc as plsc

_N = 16384
_C = 1000
_R = 128
_L = 128

_M = 2048
_NC = 2
_NS = 16
_NW = _NC * _NS
_RW = _M // _NW
_GRP = 16
_NG = _RW // _GRP
_CP = 1008
_NCH = _CP // 16

_NT = _N - _M
_BR = 512
_K = 4
_G = _NT // (_BR * _K)
_B0 = _M // _BR


def _sc_phase_a_body(logits_hbm, labels_hbm, s_hbm, v_hbm,
                     rb0, rb1, lab_v, s_v, v_v, sem0, sem1):
    bufs = (rb0, rb1)
    wid = lax.axis_index("s") * _NC + lax.axis_index("c")
    base = pl.multiple_of(wid * _RW, _RW)
    iota = lax.iota(jnp.int32, 16)

    ninf = jnp.full((16,), -jnp.inf, jnp.float32)
    for rb in bufs:
        for r in range(_GRP):
            rb[pl.ds(r * _CP + _CP - 16, 16)] = ninf

    pltpu.sync_copy(labels_hbm.at[pl.ds(base, _RW)], lab_v)

    def _dma(g, r, buf, sem):
        start = pl.multiple_of((base + g * _GRP + r) * _C, 8)
        return pltpu.make_async_copy(
            logits_hbm.at[pl.ds(start, _C)],
            bufs[buf].at[pl.ds(r * _CP, _C)],
            sem,
        )

    def _fire(g, buf, sem):
        for r in range(_GRP):
            _dma(g, r, buf, sem).start()

    def _drain(g, buf, sem):
        for r in range(_GRP):
            _dma(g, r, buf, sem).wait()

    sems = (sem0, sem1)
    _fire(0, 0, sem0)
    for g in range(_NG):
        buf = g % 2
        _drain(g, buf, sems[buf])
        if g + 1 < _NG:
            _fire(g + 1, 1 - buf, sems[1 - buf])

        def row_body(r, carry):
            s_vec, am_vec = carry
            def c_max(c, mi):
                m_l, i_l = mi
                x = bufs[buf][pl.ds(r * _CP + c * 16, 16)]
                idx = c * 16 + iota
                gt = x > m_l
                return jnp.where(gt, x, m_l), jnp.where(gt, idx, i_l)

            m_l = jnp.full((16,), -jnp.inf, jnp.float32)
            i_l = jnp.full((16,), 2**31 - 1, jnp.int32)
            m_l, i_l = lax.fori_loop(0, _NCH, c_max, (m_l, i_l))
            m_b = plsc.cummax(lax.rev(plsc.cummax(m_l), (0,)))
            i_sel = jnp.where(m_l == m_b, i_l, 2**31 - 1)
            i_first = -plsc.cummax(lax.rev(plsc.cummax(-i_sel), (0,)))

            def c_sum(c, acc):
                x = bufs[buf][pl.ds(r * _CP + c * 16, 16)]
                return acc + jnp.exp(x - m_b)

            acc = lax.fori_loop(0, _NCH, c_sum, jnp.zeros((16,), jnp.float32))
            s_r = 1.0 / plsc.cummax(lax.rev(plsc.cumsum(acc), (0,)))

            s_vec = jnp.where(iota == r, s_r, s_vec)
            am_vec = jnp.where(iota == r, i_first, am_vec)
            return s_vec, am_vec

        s_vec = jnp.zeros((16,), jnp.float32)
        am_vec = jnp.full((16,), -1, jnp.int32)
        s_vec, am_vec = lax.fori_loop(0, _GRP, row_body, (s_vec, am_vec))

        labv = lab_v[pl.ds(g * _GRP, 16)]
        l_vec = (am_vec == labv).astype(jnp.float32)
        rid = base + g * _GRP + iota
        v_vec = 2.0 * rid.astype(jnp.float32) + l_vec
        s_v[pl.ds(g * _GRP, 16)] = s_vec
        v_v[pl.ds(g * _GRP, 16)] = v_vec

    pltpu.sync_copy(s_v, s_hbm.at[pl.ds(base, _RW)])
    pltpu.sync_copy(v_v, v_hbm.at[pl.ds(base, _RW)])


def _build_sc_phase_a():
    return functools.partial(
        pl.kernel,
        mesh=plsc.VectorSubcoreMesh(core_axis_name="c", subcore_axis_name="s"),
        compiler_params=pltpu.CompilerParams(needs_layout_passes=False),
        out_type=[
            jax.ShapeDtypeStruct((_M,), jnp.float32),
            jax.ShapeDtypeStruct((_M,), jnp.float32),
        ],
        scratch_types=[
            pltpu.VMEM((_GRP * _CP,), jnp.float32),
            pltpu.VMEM((_GRP * _CP,), jnp.float32),
            pltpu.VMEM((_RW,), jnp.int32),
            pltpu.VMEM((_RW,), jnp.float32),
            pltpu.VMEM((_RW,), jnp.float32),
            pltpu.SemaphoreType.DMA,
            pltpu.SemaphoreType.DMA,
        ],
    )(_sc_phase_a_body)


def _softmax_top1_body(*refs):
    logit_refs = refs[:_K]
    labels_ref = refs[_K]
    s_refs = refs[_K + 1 : 2 * _K + 1]
    v_refs = refs[2 * _K + 1 :]
    i = pl.program_id(0)
    for k in range(_K):
        x = logit_refs[k][...]
        m = jnp.max(x, axis=1, keepdims=True)
        z = jnp.sum(jnp.exp(x - m), axis=1, keepdims=True)
        s = 1.0 / z
        col = lax.broadcasted_iota(jnp.int32, x.shape, 1)
        am = jnp.min(jnp.where(x == m, col, _C), axis=1, keepdims=True)
        lab = labels_ref[0, :, k : k + 1]
        acc = (am == lab).astype(jnp.float32)
        rid = _M + (i * _K + k) * _BR + lax.broadcasted_iota(jnp.int32, (_BR, 1), 0)
        v = 2.0 * rid.astype(jnp.float32) + acc
        s_refs[k][...] = s.reshape(1, _BR, 1)
        v_refs[k][...] = v.reshape(1, _BR, 1)


def _sort_ks_body(s_ref, v_ref, o_ref):
    s = s_ref[...]
    v = v_ref[...]
    row = lax.broadcasted_iota(jnp.int32, (_R, _L), 0)
    col = lax.broadcasted_iota(jnp.int32, (_R, _L), 1)

    def partner(x, dist, ax, lo):
        xm = jnp.roll(x, -dist, axis=ax)
        xp = jnp.roll(x, dist, axis=ax)
        return jnp.where(lo, xm, xp)

    bs = 2
    while bs <= _N:
        asc = ((col & bs) == 0) if bs < _L else ((row & (bs // _L)) == 0)
        d = bs // 2
        while d >= 1:
            if d < _L:
                ax, dist = 1, d
                lo = (col & d) == 0
            else:
                ax, dist = 0, d // _L
                lo = (row & (d // _L)) == 0
            s_p = partner(s, dist, ax, lo)
            v_p = partner(v, dist, ax, lo)
            lt = (s < s_p) | ((s == s_p) & (v < v_p))
            keep_self = (asc == lo) == lt
            s = jnp.where(keep_self, s, s_p)
            v = jnp.where(keep_self, v, v_p)
            d //= 2
        bs *= 2

    l = (v.astype(jnp.int32) & 1).astype(jnp.float32)
    dd = s - l
    tri = (row <= col).astype(jnp.float32)
    rp = lax.dot(dd, tri, precision=lax.Precision.HIGHEST)
    low = (col < row).astype(jnp.float32)
    off = lax.dot(low, rp, precision=lax.Precision.HIGHEST)[:, _L - 1 : _L]
    p = rp + off
    o_ref[...] = jnp.max(jnp.abs(p), axis=(0, 1), keepdims=True) * (1.0 / _N)


def _build_phase_a(interpret=False):
    in_specs = [
        pl.BlockSpec((_BR, _C), (lambda i, _k=k: (i * _K + _k + _B0, 0)))
        for k in range(_K)
    ] + [pl.BlockSpec((1, _BR, _K), lambda i: (i, 0, 0))]
    part_specs = [pl.BlockSpec((1, _BR, 1), lambda i: (i, 0, 0)) for _ in range(2 * _K)]
    part_shapes = [jax.ShapeDtypeStruct((_G, _BR, 1), jnp.float32) for _ in range(2 * _K)]
    return pl.pallas_call(
        _softmax_top1_body,
        grid=(_G,),
        in_specs=in_specs,
        out_specs=part_specs,
        out_shape=part_shapes,
        interpret=interpret,
    )


def _build_phase_b(interpret=False):
    return pl.pallas_call(
        _sort_ks_body,
        out_shape=jax.ShapeDtypeStruct((1, 1), jnp.float32),
        interpret=interpret,
    )


def _assemble(parts):
    return jnp.stack([p.reshape(_G, _BR) for p in parts], axis=1).reshape(-1, _L)


def kernel(logits, labels):
    labels_i32 = labels.astype(jnp.int32)
    sc_s, sc_v = _build_sc_phase_a()(logits.reshape(-1), labels_i32[:_M])
    labels3 = labels_i32[_M:].reshape(_G, _K, _BR).transpose(0, 2, 1)
    outs = _build_phase_a()(*([logits] * _K), labels3)
    s2 = jnp.concatenate([sc_s.reshape(-1, _L), _assemble(outs[:_K])], axis=0)
    v2 = jnp.concatenate([sc_v.reshape(-1, _L), _assemble(outs[_K:])], axis=0)
    out = _build_phase_b()(s2, v2)
    return out[0, 0]
